# Initial kernel scaffold; baseline (speedup 1.0000x reference)
import jax
import jax.numpy as jnp
from jax.experimental import pallas as pl


def kernel(node_f, edge_index):
    raise NotImplementedError("write your pallas kernel here")


# speedup vs baseline: 4.8026x; 4.8026x over previous
import functools

import jax
import jax.numpy as jnp
from jax import lax
from jax.experimental import pallas as pl
from jax.experimental.pallas import tpu as pltpu
from jax.experimental.pallas import tpu_sc as plsc

N = 10000
D = 128
E = 320000

NC, NS, L = 2, 16, 16
NW = NC * NS
CH = 128
NCHUNK = E // CH
BASE_CHUNKS = NCHUNK // NW
EXTRA = NCHUNK - BASE_CHUNKS * NW
ROWS_MAIN = 624
ROWS_TAIL = N - ROWS_MAIN * NS


def _fill_rows(ref, nrows, ncols, value):
    per_row = ncols // L

    def body(k, _):
        i = k // per_row
        j = k % per_row
        ref[i, pl.ds(j * L, L)] = jnp.full((L,), value, ref.dtype)
        return 0

    lax.fori_loop(0, nrows * per_row, body, 0)


def _zero_span(zeros_v, dst, start, nrows):
    full, rem = nrows // CH, nrows % CH
    for k in range(full):
        pltpu.sync_copy(zeros_v, dst.at[pl.ds(start + k * CH, CH)])
    if rem:
        pltpu.sync_copy(zeros_v.at[pl.ds(0, rem)],
                        dst.at[pl.ds(start + full * CH, rem)])


def _zero_tile_rows(zeros_v, dst, sid):
    _zero_span(zeros_v, dst, sid * ROWS_MAIN, ROWS_MAIN)

    @pl.when(sid == NS - 1)
    def _():
        _zero_span(zeros_v, dst, ROWS_MAIN * NS, ROWS_TAIL)


def _write_tile_rows(src, dst_core, sid):
    r0 = sid * ROWS_MAIN
    pltpu.sync_copy(src.at[pl.ds(r0, ROWS_MAIN)],
                    dst_core.at[pl.ds(r0, ROWS_MAIN)])

    @pl.when(sid == NS - 1)
    def _():
        pltpu.sync_copy(src.at[pl.ds(ROWS_MAIN * NS, ROWS_TAIL)],
                        dst_core.at[pl.ds(ROWS_MAIN * NS, ROWS_TAIL)])


_sc_mesh = plsc.VectorSubcoreMesh(core_axis_name="c", subcore_axis_name="s")


def _fill_1d(ref, n, value):
    for j in range(n // L):
        ref[pl.ds(j * L, L)] = jnp.full((L,), value, ref.dtype)


@functools.partial(
    pl.kernel,
    out_type=jax.ShapeDtypeStruct((NC * 2 * N,), jnp.float32),
    mesh=_sc_mesh,
    scratch_types=(
        pltpu.VMEM_SHARED((N,), jnp.float32),
        pltpu.VMEM_SHARED((N,), jnp.float32),
        pltpu.VMEM((CH,), jnp.float32),
        pltpu.VMEM((CH,), jnp.float32),
        pltpu.VMEM((CH,), jnp.int32),
        pltpu.VMEM((ROWS_MAIN + ROWS_TAIL,), jnp.float32),
    ),
)
def _degree_kernel(src_hbm, dst_hbm, cnt_out,
                   cnt_src, cnt_dst, ones_v, zeros_v, idx_v, stage_v):
    cid = lax.axis_index("c")
    sid = lax.axis_index("s")
    wid = sid * NC + cid

    _fill_1d(ones_v, CH, 1.0)
    _fill_1d(zeros_v, CH, 0.0)
    _zero_tile_rows(zeros_v, cnt_src, sid)
    _zero_tile_rows(zeros_v, cnt_dst, sid)
    plsc.subcore_barrier()

    def do_chunk(c):
        base = c * CH
        pltpu.sync_copy(src_hbm.at[pl.ds(base, CH)], idx_v)
        pltpu.sync_copy(ones_v, cnt_src.at[idx_v], add=True)
        pltpu.sync_copy(dst_hbm.at[pl.ds(base, CH)], idx_v)
        pltpu.sync_copy(ones_v, cnt_dst.at[idx_v], add=True)

    def body(i, _):
        do_chunk(i * NW + wid)
        return 0

    lax.fori_loop(0, BASE_CHUNKS, body, 0)

    @pl.when(wid < EXTRA)
    def _():
        do_chunk(BASE_CHUNKS * NW + wid)

    plsc.subcore_barrier()

    def readout(cnt, out_base):
        r0 = sid * ROWS_MAIN
        pltpu.sync_copy(cnt.at[pl.ds(r0, ROWS_MAIN)],
                        stage_v.at[pl.ds(0, ROWS_MAIN)])
        pltpu.sync_copy(stage_v.at[pl.ds(0, ROWS_MAIN)],
                        cnt_out.at[pl.ds(out_base + r0, ROWS_MAIN)])

        @pl.when(sid == NS - 1)
        def _():
            t0 = ROWS_MAIN * NS
            pltpu.sync_copy(cnt.at[pl.ds(t0, ROWS_TAIL)],
                            stage_v.at[pl.ds(0, ROWS_TAIL)])
            pltpu.sync_copy(stage_v.at[pl.ds(0, ROWS_TAIL)],
                            cnt_out.at[pl.ds(out_base + t0, ROWS_TAIL)])

    readout(cnt_src, cid * 2 * N)
    readout(cnt_dst, cid * 2 * N + N)


@functools.partial(
    pl.kernel,
    out_type=jax.ShapeDtypeStruct((NC, N, D), jnp.float32),
    mesh=_sc_mesh,
    scratch_types=(
        pltpu.VMEM_SHARED((N, D), jnp.float32),
        pltpu.VMEM((CH, D), jnp.float32),
        pltpu.VMEM((CH,), jnp.int32),
        pltpu.VMEM((CH,), jnp.int32),
        pltpu.SemaphoreType.DMA,
    ),
)
def _aggregate_kernel(h_hbm, src_hbm, dst_hbm, part_out,
                      acc, rows_v, sidx_v, didx_v, sem):
    cid = lax.axis_index("c")
    sid = lax.axis_index("s")
    wid = sid * NC + cid

    _fill_rows(rows_v, CH, D, 0.0)
    _zero_tile_rows(rows_v, acc, sid)
    plsc.subcore_barrier()

    def do_chunk(c):
        base = c * CH
        pltpu.sync_copy(src_hbm.at[pl.ds(base, CH)], sidx_v)
        pltpu.sync_copy(dst_hbm.at[pl.ds(base, CH)], didx_v)
        pltpu.async_copy(h_hbm.at[sidx_v], rows_v, sem).wait()
        pltpu.sync_copy(rows_v, acc.at[didx_v], add=True)

    def body(i, _):
        do_chunk(i * NW + wid)
        return 0

    lax.fori_loop(0, BASE_CHUNKS, body, 0)

    @pl.when(wid < EXTRA)
    def _():
        do_chunk(BASE_CHUNKS * NW + wid)

    plsc.subcore_barrier()
    _write_tile_rows(acc, part_out.at[cid], sid)


_BLK = 1000


def _scale_body(node_ref, cnt_ref, h_ref):
    deg = cnt_ref[0] + cnt_ref[1]
    h_ref[...] = node_ref[...] * jax.lax.rsqrt(jnp.maximum(deg, 1.0))


_scale_kernel = pl.pallas_call(
    _scale_body,
    grid=(N // _BLK,),
    in_specs=[
        pl.BlockSpec((_BLK, D), lambda i: (i, 0)),
        pl.BlockSpec((NC, _BLK, 1), lambda i: (0, i, 0)),
    ],
    out_specs=pl.BlockSpec((_BLK, D), lambda i: (i, 0)),
    out_shape=jax.ShapeDtypeStruct((N, D), jnp.float32),
)


def _combine_body(part_ref, cnt_ref, out_ref):
    deg = cnt_ref[0] + cnt_ref[1]
    agg = part_ref[0] + part_ref[1]
    out_ref[...] = agg * jax.lax.rsqrt(jnp.maximum(deg, 1.0))


_combine_kernel = pl.pallas_call(
    _combine_body,
    grid=(N // _BLK,),
    in_specs=[
        pl.BlockSpec((NC, _BLK, D), lambda i: (0, i, 0)),
        pl.BlockSpec((NC, _BLK, 1), lambda i: (0, i, 0)),
    ],
    out_specs=pl.BlockSpec((_BLK, D), lambda i: (i, 0)),
    out_shape=jax.ShapeDtypeStruct((N, D), jnp.float32),
)


def kernel(node_f, edge_index):
    ei = edge_index.astype(jnp.int32)
    src = ei[0]
    dst = ei[1]
    cnt = _degree_kernel(src, dst).reshape(NC, 2, N)
    cnt_src = cnt[:, 0, :].reshape(NC, N, 1)
    cnt_dst = cnt[:, 1, :].reshape(NC, N, 1)
    h = _scale_kernel(node_f, cnt_src)
    partials = _aggregate_kernel(h, src, dst)
    return _combine_kernel(partials, cnt_dst)


# speedup vs baseline: 10.3277x; 2.1504x over previous
import functools

import jax
import jax.numpy as jnp
from jax import lax
from jax.experimental import pallas as pl
from jax.experimental.pallas import tpu as pltpu
from jax.experimental.pallas import tpu_sc as plsc

N = 10000
D = 128
E = 320000

NC, NS, L = 2, 16, 16
NW = NC * NS
CH = 128

NPAD = 16
N_P = N + NPAD
E_P = 327680
PAD_E = E_P - E
NCHUNK = E_P // CH
TILE_CHUNKS = NCHUNK // NW

ROWS_MAIN = 624
ROWS_TAIL = N - ROWS_MAIN * NS
ZROWS_TAIL = N_P - ROWS_MAIN * NS


def _fill_rows(ref, nrows, ncols, value):
    per_row = ncols // L

    def body(k, _):
        i = k // per_row
        j = k % per_row
        ref[i, pl.ds(j * L, L)] = jnp.full((L,), value, ref.dtype)
        return 0

    lax.fori_loop(0, nrows * per_row, body, 0)


def _fill_1d(ref, n, value):
    for j in range(n // L):
        ref[pl.ds(j * L, L)] = jnp.full((L,), value, ref.dtype)


def _zero_span(zeros_v, dst, start, nrows, zrows):
    full, rem = nrows // zrows, nrows % zrows
    for k in range(full):
        pltpu.sync_copy(zeros_v, dst.at[pl.ds(start + k * zrows, zrows)])
    if rem:
        pltpu.sync_copy(zeros_v.at[pl.ds(0, rem)],
                        dst.at[pl.ds(start + full * zrows, rem)])


def _zero_tile_rows(zeros_v, dst, sid, zrows):
    _zero_span(zeros_v, dst, sid * ROWS_MAIN, ROWS_MAIN, zrows)

    @pl.when(sid == NS - 1)
    def _():
        _zero_span(zeros_v, dst, ROWS_MAIN * NS, ZROWS_TAIL, zrows)


_sc_mesh = plsc.VectorSubcoreMesh(core_axis_name="c", subcore_axis_name="s")


@functools.partial(
    pl.kernel,
    out_type=jax.ShapeDtypeStruct((NC * 2 * N,), jnp.float32),
    mesh=_sc_mesh,
    scratch_types=(
        pltpu.VMEM_SHARED((N_P,), jnp.float32),
        pltpu.VMEM_SHARED((N_P,), jnp.float32),
        pltpu.VMEM((TILE_CHUNKS, CH), jnp.int32),
        pltpu.VMEM((TILE_CHUNKS, CH), jnp.int32),
        pltpu.VMEM((CH,), jnp.float32),
        pltpu.VMEM((ROWS_MAIN + ZROWS_TAIL,), jnp.float32),
        pltpu.SemaphoreType.DMA,
    ),
)
def _degree_kernel(src_hbm, dst_hbm, cnt_out,
                   cnt_src, cnt_dst, sidx_all, didx_all, ones_v, zeros_v, sem):
    cid = lax.axis_index("c")
    sid = lax.axis_index("s")
    wid = sid * NC + cid

    pltpu.sync_copy(src_hbm.at[pl.ds(wid * TILE_CHUNKS, TILE_CHUNKS)], sidx_all)
    pltpu.sync_copy(dst_hbm.at[pl.ds(wid * TILE_CHUNKS, TILE_CHUNKS)], didx_all)
    _fill_1d(ones_v, CH, 1.0)
    _fill_1d(zeros_v, ROWS_MAIN + ZROWS_TAIL, 0.0)
    _zero_tile_rows(zeros_v, cnt_src, sid, ROWS_MAIN + ZROWS_TAIL)
    _zero_tile_rows(zeros_v, cnt_dst, sid, ROWS_MAIN + ZROWS_TAIL)
    plsc.subcore_barrier()

    def issue(i, _):
        pltpu.async_copy(ones_v, cnt_src.at[sidx_all.at[i]], sem, add=True)
        pltpu.async_copy(ones_v, cnt_dst.at[didx_all.at[i]], sem, add=True)
        return 0

    lax.fori_loop(0, TILE_CHUNKS, issue, 0)

    def drain(i, _):
        pltpu.make_async_copy(ones_v, cnt_src.at[sidx_all.at[i]], sem).wait()
        pltpu.make_async_copy(ones_v, cnt_dst.at[didx_all.at[i]], sem).wait()
        return 0

    lax.fori_loop(0, TILE_CHUNKS, drain, 0)

    plsc.subcore_barrier()

    def readout(cnt, out_base):
        r0 = sid * ROWS_MAIN
        pltpu.sync_copy(cnt.at[pl.ds(r0, ROWS_MAIN)],
                        zeros_v.at[pl.ds(0, ROWS_MAIN)])
        pltpu.sync_copy(zeros_v.at[pl.ds(0, ROWS_MAIN)],
                        cnt_out.at[pl.ds(out_base + r0, ROWS_MAIN)])

        @pl.when(sid == NS - 1)
        def _():
            t0 = ROWS_MAIN * NS
            pltpu.sync_copy(cnt.at[pl.ds(t0, ROWS_TAIL)],
                            zeros_v.at[pl.ds(0, ROWS_TAIL)])
            pltpu.sync_copy(zeros_v.at[pl.ds(0, ROWS_TAIL)],
                            cnt_out.at[pl.ds(out_base + t0, ROWS_TAIL)])

    readout(cnt_src, cid * 2 * N)
    readout(cnt_dst, cid * 2 * N + N)


NSTAGE = 2
SCHUNK = TILE_CHUNKS // NSTAGE


@functools.partial(
    pl.kernel,
    out_type=jax.ShapeDtypeStruct((NC, N, D), jnp.float32),
    mesh=_sc_mesh,
    scratch_types=(
        pltpu.VMEM_SHARED((N_P, D), jnp.float32),
        pltpu.VMEM((SCHUNK, CH), jnp.int32),
        pltpu.VMEM((SCHUNK, CH), jnp.int32),
        pltpu.VMEM((CH, D), jnp.float32),
        pltpu.VMEM((CH, D), jnp.float32),
        pltpu.SemaphoreType.DMA,
        pltpu.SemaphoreType.DMA,
        pltpu.SemaphoreType.DMA,
        pltpu.SemaphoreType.DMA,
    ),
)
def _aggregate_kernel(h_hbm, src_hbm, dst_hbm, part_out,
                      acc, sidx_all, didx_all, r0_v, r1_v, g0, g1, s0, s1):
    cid = lax.axis_index("c")
    sid = lax.axis_index("s")
    wid = sid * NC + cid
    rows = (r0_v, r1_v)
    gsem = (g0, g1)
    ssem = (s0, s1)

    _fill_rows(r0_v, CH, D, 0.0)
    _zero_tile_rows(r0_v, acc, sid, CH)
    plsc.subcore_barrier()

    def start_gather(j, b):
        pltpu.async_copy(h_hbm.at[sidx_all.at[j]], rows[b], gsem[b])

    def wait_gather(b):
        pltpu.make_async_copy(h_hbm.at[sidx_all.at[0]], rows[b], gsem[b]).wait()

    def start_scatter(j, b):
        pltpu.async_copy(rows[b], acc.at[didx_all.at[j]], ssem[b], add=True)

    def wait_scatter(b):
        pltpu.make_async_copy(rows[b], acc.at[didx_all.at[0]], ssem[b]).wait()

    def step(j, b, prefetch):
        wait_gather(b)
        start_scatter(j, b)
        if prefetch:
            wait_scatter(b)
            start_gather(j + 2, b)

    for stage in range(NSTAGE):
        base = wid * TILE_CHUNKS + stage * SCHUNK
        pltpu.sync_copy(src_hbm.at[pl.ds(base, SCHUNK)], sidx_all)
        pltpu.sync_copy(dst_hbm.at[pl.ds(base, SCHUNK)], didx_all)
        start_gather(0, 0)
        start_gather(1, 1)

        def body(g, _):
            step(2 * g, 0, prefetch=True)
            step(2 * g + 1, 1, prefetch=True)
            return 0

        lax.fori_loop(0, SCHUNK // 2 - 1, body, 0)
        step(SCHUNK - 2, 0, prefetch=False)
        step(SCHUNK - 1, 1, prefetch=False)
        wait_scatter(0)
        wait_scatter(1)

    plsc.subcore_barrier()
    r0 = sid * ROWS_MAIN
    pltpu.sync_copy(acc.at[pl.ds(r0, ROWS_MAIN)],
                    part_out.at[cid, pl.ds(r0, ROWS_MAIN)])

    @pl.when(sid == NS - 1)
    def _():
        t0 = ROWS_MAIN * NS
        pltpu.sync_copy(acc.at[pl.ds(t0, ROWS_TAIL)],
                        part_out.at[cid, pl.ds(t0, ROWS_TAIL)])


def _scale_body(node_ref, cnt_ref, h_ref):
    deg = cnt_ref[0] + cnt_ref[1]
    h_ref[pl.ds(0, N), :] = node_ref[...] * jax.lax.rsqrt(jnp.maximum(deg, 1.0))
    h_ref[pl.ds(N, NPAD), :] = jnp.zeros((NPAD, D), jnp.float32)


_scale_kernel = pl.pallas_call(
    _scale_body,
    out_shape=jax.ShapeDtypeStruct((N_P, D), jnp.float32),
)


def _combine_body(part_ref, cnt_ref, out_ref):
    deg = cnt_ref[0] + cnt_ref[1]
    agg = part_ref[0] + part_ref[1]
    out_ref[...] = agg * jax.lax.rsqrt(jnp.maximum(deg, 1.0))


_combine_kernel = pl.pallas_call(
    _combine_body,
    out_shape=jax.ShapeDtypeStruct((N, D), jnp.float32),
)


def kernel(node_f, edge_index):
    ei = edge_index.astype(jnp.int32)
    pad = N + (jnp.arange(PAD_E, dtype=jnp.int32) % NPAD)
    src = jnp.concatenate([ei[0], pad]).reshape(NCHUNK, CH)
    dst = jnp.concatenate([ei[1], pad]).reshape(NCHUNK, CH)
    cnt = _degree_kernel(src, dst).reshape(NC, 2, N)
    cnt_src = cnt[:, 0, :].reshape(NC, N, 1)
    cnt_dst = cnt[:, 1, :].reshape(NC, N, 1)
    h = _scale_kernel(node_f, cnt_src)
    partials = _aggregate_kernel(h, src, dst)
    return _combine_kernel(partials, cnt_dst)


# speedup vs baseline: 10.4330x; 1.0102x over previous
import functools

import jax
import jax.numpy as jnp
from jax import lax
from jax.experimental import pallas as pl
from jax.experimental.pallas import tpu as pltpu
from jax.experimental.pallas import tpu_sc as plsc

N = 10000
D = 128
E = 320000

NC, NS, L = 2, 16, 16
NW = NC * NS
CH = 128

NPAD = 16
N_P = N + NPAD
E_P = 327680
PAD_E = E_P - E
NCHUNK = E_P // CH
TILE_CHUNKS = NCHUNK // NW

ROWS_MAIN = 624
ROWS_TAIL = N - ROWS_MAIN * NS
ZROWS_TAIL = N_P - ROWS_MAIN * NS


def _fill_rows(ref, nrows, ncols, value):
    per_row = ncols // L

    def body(k, _):
        i = k // per_row
        j = k % per_row
        ref[i, pl.ds(j * L, L)] = jnp.full((L,), value, ref.dtype)
        return 0

    lax.fori_loop(0, nrows * per_row, body, 0)


def _fill_1d(ref, n, value):
    for j in range(n // L):
        ref[pl.ds(j * L, L)] = jnp.full((L,), value, ref.dtype)


def _zero_span(zeros_v, dst, start, nrows, zrows):
    full, rem = nrows // zrows, nrows % zrows
    for k in range(full):
        pltpu.sync_copy(zeros_v, dst.at[pl.ds(start + k * zrows, zrows)])
    if rem:
        pltpu.sync_copy(zeros_v.at[pl.ds(0, rem)],
                        dst.at[pl.ds(start + full * zrows, rem)])


def _zero_tile_rows(zeros_v, dst, sid, zrows):
    _zero_span(zeros_v, dst, sid * ROWS_MAIN, ROWS_MAIN, zrows)

    @pl.when(sid == NS - 1)
    def _():
        _zero_span(zeros_v, dst, ROWS_MAIN * NS, ZROWS_TAIL, zrows)


_sc_mesh = plsc.VectorSubcoreMesh(core_axis_name="c", subcore_axis_name="s")


@functools.partial(
    pl.kernel,
    out_type=jax.ShapeDtypeStruct((NC * 2 * N,), jnp.float32),
    mesh=_sc_mesh,
    scratch_types=(
        pltpu.VMEM_SHARED((N_P,), jnp.float32),
        pltpu.VMEM_SHARED((N_P,), jnp.float32),
        pltpu.VMEM((TILE_CHUNKS, CH), jnp.int32),
        pltpu.VMEM((TILE_CHUNKS, CH), jnp.int32),
        pltpu.VMEM((CH,), jnp.float32),
        pltpu.VMEM((ROWS_MAIN + ZROWS_TAIL,), jnp.float32),
        pltpu.SemaphoreType.DMA,
    ),
)
def _degree_kernel(src_hbm, dst_hbm, cnt_out,
                   cnt_src, cnt_dst, sidx_all, didx_all, ones_v, zeros_v, sem):
    cid = lax.axis_index("c")
    sid = lax.axis_index("s")
    wid = sid * NC + cid

    pltpu.sync_copy(src_hbm.at[pl.ds(wid * TILE_CHUNKS, TILE_CHUNKS)], sidx_all)
    pltpu.sync_copy(dst_hbm.at[pl.ds(wid * TILE_CHUNKS, TILE_CHUNKS)], didx_all)
    _fill_1d(ones_v, CH, 1.0)
    _fill_1d(zeros_v, ROWS_MAIN + ZROWS_TAIL, 0.0)
    _zero_tile_rows(zeros_v, cnt_src, sid, ROWS_MAIN + ZROWS_TAIL)
    _zero_tile_rows(zeros_v, cnt_dst, sid, ROWS_MAIN + ZROWS_TAIL)
    plsc.subcore_barrier()

    def issue(i, _):
        pltpu.async_copy(ones_v, cnt_src.at[sidx_all.at[i]], sem, add=True)
        pltpu.async_copy(ones_v, cnt_dst.at[didx_all.at[i]], sem, add=True)
        return 0

    lax.fori_loop(0, TILE_CHUNKS, issue, 0)

    def drain(i, _):
        pltpu.make_async_copy(ones_v, cnt_src.at[sidx_all.at[i]], sem).wait()
        pltpu.make_async_copy(ones_v, cnt_dst.at[didx_all.at[i]], sem).wait()
        return 0

    lax.fori_loop(0, TILE_CHUNKS, drain, 0)

    plsc.subcore_barrier()

    def readout(cnt, out_base):
        r0 = sid * ROWS_MAIN
        pltpu.sync_copy(cnt.at[pl.ds(r0, ROWS_MAIN)],
                        zeros_v.at[pl.ds(0, ROWS_MAIN)])
        pltpu.sync_copy(zeros_v.at[pl.ds(0, ROWS_MAIN)],
                        cnt_out.at[pl.ds(out_base + r0, ROWS_MAIN)])

        @pl.when(sid == NS - 1)
        def _():
            t0 = ROWS_MAIN * NS
            pltpu.sync_copy(cnt.at[pl.ds(t0, ROWS_TAIL)],
                            zeros_v.at[pl.ds(0, ROWS_TAIL)])
            pltpu.sync_copy(zeros_v.at[pl.ds(0, ROWS_TAIL)],
                            cnt_out.at[pl.ds(out_base + t0, ROWS_TAIL)])

    readout(cnt_src, cid * 2 * N)
    readout(cnt_dst, cid * 2 * N + N)


NSTAGE = 2
SCHUNK = TILE_CHUNKS // NSTAGE


@functools.partial(
    pl.kernel,
    out_type=jax.ShapeDtypeStruct((NC, N, D), jnp.float32),
    mesh=_sc_mesh,
    scratch_types=(
        pltpu.VMEM_SHARED((N_P, D), jnp.float32),
        pltpu.VMEM((SCHUNK, CH), jnp.int32),
        pltpu.VMEM((SCHUNK, CH), jnp.int32),
        pltpu.VMEM((CH, D), jnp.float32),
        pltpu.VMEM((CH, D), jnp.float32),
        pltpu.VMEM((48, D), jnp.float32),
        pltpu.SemaphoreType.DMA,
        pltpu.SemaphoreType.DMA,
        pltpu.SemaphoreType.DMA,
        pltpu.SemaphoreType.DMA,
    ),
)
def _aggregate_kernel(h_hbm, src_hbm, dst_hbm, part_out,
                      acc, sidx_all, didx_all, r0_v, r1_v, zeros_v,
                      g0, g1, s0, s1):
    cid = lax.axis_index("c")
    sid = lax.axis_index("s")
    wid = sid * NC + cid
    rows = (r0_v, r1_v)
    gsem = (g0, g1)
    ssem = (s0, s1)

    def start_gather(j, b):
        pltpu.async_copy(h_hbm.at[sidx_all.at[j]], rows[b], gsem[b])

    def wait_gather(b):
        pltpu.make_async_copy(h_hbm.at[sidx_all.at[0]], rows[b], gsem[b]).wait()

    def start_scatter(j, b):
        pltpu.async_copy(rows[b], acc.at[didx_all.at[j]], ssem[b], add=True)

    def wait_scatter(b):
        pltpu.make_async_copy(rows[b], acc.at[didx_all.at[0]], ssem[b]).wait()

    def step(j, b, prefetch):
        wait_gather(b)
        start_scatter(j, b)
        if prefetch:
            wait_scatter(b)
            start_gather(j + 2, b)

    for stage in range(NSTAGE):
        base = wid * TILE_CHUNKS + stage * SCHUNK
        pltpu.sync_copy(src_hbm.at[pl.ds(base, SCHUNK)], sidx_all)
        pltpu.sync_copy(dst_hbm.at[pl.ds(base, SCHUNK)], didx_all)
        start_gather(0, 0)
        start_gather(1, 1)

        if stage == 0:
            _fill_rows(zeros_v, 48, D, 0.0)
            _zero_tile_rows(zeros_v, acc, sid, 48)
            plsc.subcore_barrier()

        def body(g, _):
            step(2 * g, 0, prefetch=True)
            step(2 * g + 1, 1, prefetch=True)
            return 0

        lax.fori_loop(0, SCHUNK // 2 - 1, body, 0)
        step(SCHUNK - 2, 0, prefetch=False)
        step(SCHUNK - 1, 1, prefetch=False)
        wait_scatter(0)
        wait_scatter(1)

    plsc.subcore_barrier()
    r0 = sid * ROWS_MAIN
    pltpu.sync_copy(acc.at[pl.ds(r0, ROWS_MAIN)],
                    part_out.at[cid, pl.ds(r0, ROWS_MAIN)])

    @pl.when(sid == NS - 1)
    def _():
        t0 = ROWS_MAIN * NS
        pltpu.sync_copy(acc.at[pl.ds(t0, ROWS_TAIL)],
                        part_out.at[cid, pl.ds(t0, ROWS_TAIL)])


def _scale_body(node_ref, cnt_ref, h_ref):
    deg = cnt_ref[0] + cnt_ref[1]
    h_ref[pl.ds(0, N), :] = node_ref[...] * jax.lax.rsqrt(jnp.maximum(deg, 1.0))
    h_ref[pl.ds(N, NPAD), :] = jnp.zeros((NPAD, D), jnp.float32)


_scale_kernel = pl.pallas_call(
    _scale_body,
    out_shape=jax.ShapeDtypeStruct((N_P, D), jnp.float32),
)


def _combine_body(part_ref, cnt_ref, out_ref):
    deg = cnt_ref[0] + cnt_ref[1]
    agg = part_ref[0] + part_ref[1]
    out_ref[...] = agg * jax.lax.rsqrt(jnp.maximum(deg, 1.0))


_BLK = 1000

_combine_kernel = pl.pallas_call(
    _combine_body,
    grid=(N // _BLK,),
    in_specs=[
        pl.BlockSpec((NC, _BLK, D), lambda i: (0, i, 0)),
        pl.BlockSpec((NC, _BLK, 1), lambda i: (0, i, 0)),
    ],
    out_specs=pl.BlockSpec((_BLK, D), lambda i: (i, 0)),
    out_shape=jax.ShapeDtypeStruct((N, D), jnp.float32),
)


def kernel(node_f, edge_index):
    ei = edge_index.astype(jnp.int32)
    pad = N + (jnp.arange(PAD_E, dtype=jnp.int32) % NPAD)
    src = jnp.concatenate([ei[0], pad]).reshape(NCHUNK, CH)
    dst = jnp.concatenate([ei[1], pad]).reshape(NCHUNK, CH)
    cnt = _degree_kernel(src, dst).reshape(NC, 2, N)
    cnt_src = cnt[:, 0, :].reshape(NC, N, 1)
    cnt_dst = cnt[:, 1, :].reshape(NC, N, 1)
    h = _scale_kernel(node_f, cnt_src)
    partials = _aggregate_kernel(h, src, dst)
    return _combine_kernel(partials, cnt_dst)
